# baseline (speedup 1.0000x reference)
import functools

import jax
import jax.numpy as jnp
import numpy as np
from jax import lax
from jax.experimental import pallas as pl
from jax.experimental.pallas import tpu as pltpu
from jax.experimental.pallas import tpu_sc as plsc
from jax._src.random.threefry2x32 import threefry2x32_p

NSEG = 16384
N = NSEG * 1000

NC = 2
NS = 16
NW = NC * NS
L = 16

CHUNK = N // NW
BLK = 12800
SPAN = 256
NBLK = CHUNK // BLK
NSPAN = BLK // SPAN
NEG_INF = float("-inf")

G = 4
GE = N // G
GCHUNK = GE // NW
GNBLK = GCHUNK // BLK


def _wid():
    return lax.axis_index("c") * NS + lax.axis_index("s")


def _mesh():
    return plsc.VectorSubcoreMesh(
        core_axis_name="c", subcore_axis_name="s", num_cores=NC, num_subcores=NS
    )


def _make_seg_max_body(gbase, first):
    def body(*args):
        if first:
            (y_hbm, idx_hbm, mp_hbm,
             y0_v, y1_v, i0_v, i1_v, tab_v, yscr_v, sy0, sy1, si0, si1) = args
            mpin_hbm = None
        else:
            (y_hbm, idx_hbm, mpin_hbm, mp_hbm,
             y0_v, y1_v, i0_v, i1_v, tab_v, yscr_v, sy0, sy1, si0, si1) = args
        wid = _wid()
        base = wid * GCHUNK
        gidx = gbase + wid * GCHUNK
        lane = lax.iota(jnp.int32, L)
        ybuf = (y0_v, y1_v)
        ibuf = (i0_v, i1_v)
        ysem = (sy0, sy1)
        isem = (si0, si1)

        if first:
            def init(i, c):
                tab_v[pl.ds(i * L, L)] = jnp.full((L,), NEG_INF, jnp.float32)
                return c

            lax.fori_loop(0, NSEG // L + 1, init, 0)
        else:
            pltpu.sync_copy(
                mpin_hbm.at[pl.ds(wid * NSEG, NSEG)], tab_v.at[pl.ds(0, NSEG)]
            )
            tab_v[pl.ds(NSEG, L)] = jnp.full((L,), NEG_INF, jnp.float32)

        def start_in(b, p):
            pltpu.async_copy(y_hbm.at[pl.ds(base + b * BLK, BLK)], ybuf[p], ysem[p])
            pltpu.async_copy(idx_hbm.at[pl.ds(gidx + b * BLK, BLK)], ibuf[p], isem[p])

        def wait_in(b, p):
            pltpu.make_async_copy(
                y_hbm.at[pl.ds(base + b * BLK, BLK)], ybuf[p], ysem[p]
            ).wait()
            pltpu.make_async_copy(
                idx_hbm.at[pl.ds(gidx + b * BLK, BLK)], ibuf[p], isem[p]
            ).wait()

        def rmw(ixvec, val_vec, mask):
            t = plsc.load_gather(tab_v, [ixvec])
            plsc.store_scatter(tab_v, [ixvec], jnp.maximum(t, val_vec), mask=mask)

        def flush(cs, acc):
            m = jnp.max(acc)
            rmw(jnp.full((L,), cs, jnp.int32), jnp.full((L,), m, jnp.float32), lane == 0)

        def compute(p, cs, acc):
            yb = ybuf[p]
            ib = ibuf[p]

            def kogge(vb):
                ix = ib[pl.ds(vb, L)]
                yy = yb[pl.ds(vb, L)]
                vbv = jnp.full((L,), vb, jnp.int32)
                for d in (1, 2, 4, 8):
                    ok0 = lane >= d
                    yscr_v[...] = yy
                    ys = plsc.load_gather(yscr_v, [lane - d], mask=ok0)
                    ixs = plsc.load_gather(ib, [vbv + (lane - d)], mask=ok0)
                    ok = ok0 & (ix == ixs)
                    yy = jnp.where(ok, jnp.maximum(yy, ys), yy)
                nmask = lane < (L - 1)
                ixn = plsc.load_gather(ib, [vbv + (lane + 1)], mask=nmask)
                end = (lane == L - 1) | (nmask & (ix != ixn))
                rmw(ix, yy, end)

            def span(sp, carry):
                cs, acc = carry
                sbl = sp * SPAN
                pos = jnp.full((L,), sbl, jnp.int32) + lane * L
                starts = plsc.load_gather(ib, [pos])
                ends = plsc.load_gather(ib, [pos + (L - 1)])
                sa = starts[0]
                sz = ends[L - 1]

                def uniform_case(cs, acc):
                    v = yb[pl.ds(sbl, L)]
                    for i in range(1, SPAN // L):
                        v = jnp.maximum(v, yb[pl.ds(sbl + i * L, L)])

                    def same():
                        return cs, jnp.maximum(acc, v)

                    def diff():
                        flush(cs, acc)
                        return sa, v

                    return lax.cond(sa == cs, same, diff)

                def slow_case(cs, acc):
                    flush(cs, acc)
                    kb = plsc.all_reduce_ffs(ends != jnp.full((L,), sa, jnp.int32))[0]
                    kcr = plsc.all_reduce_ffs(
                        lax.rev(starts, (0,)) != jnp.full((L,), sz, jnp.int32)
                    )[0]
                    kc = (L - 1) - kcr

                    def lstep(k, a):
                        return jnp.maximum(a, yb[pl.ds(sbl + k * L, L)])

                    lacc = lax.fori_loop(
                        0, kb, lstep, jnp.full((L,), NEG_INF, jnp.float32)
                    )
                    flush(sa, lacc)

                    def kstep(k, c):
                        kogge(sbl + k * L)
                        return c

                    lax.fori_loop(kb, kc + 1, kstep, 0)
                    racc = lax.fori_loop(
                        kc + 1, SPAN // L, lstep, jnp.full((L,), NEG_INF, jnp.float32)
                    )
                    return sz, racc

                return lax.cond(sa == sz, uniform_case, slow_case, cs, acc)

            return lax.fori_loop(0, NSPAN, span, (cs, acc))

        start_in(0, 0)

        def outer(b2, carry):
            cs, acc = carry
            for p in (0, 1):
                b = b2 * 2 + p
                pl.when(b + 1 < GNBLK)(lambda: start_in(b + 1, 1 - p))
                wait_in(b, p)
                cs, acc = compute(p, cs, acc)
            return cs, acc

        cs0 = jnp.int32(NSEG)
        acc0 = jnp.full((L,), NEG_INF, jnp.float32)
        cs, acc = lax.fori_loop(0, GNBLK // 2, outer, (cs0, acc0))
        flush(cs, acc)
        pltpu.sync_copy(tab_v.at[pl.ds(0, NSEG)], mp_hbm.at[pl.ds(wid * NSEG, NSEG)])

    return body


def _merge_body(mp_hbm, m_hbm, tmp_v, acc_v):
    wid = _wid()
    segs = NSEG // NW
    sbase = wid * segs

    def initb(j, c):
        acc_v[pl.ds(j * L, L)] = jnp.full((L,), NEG_INF, jnp.float32)
        return c

    lax.fori_loop(0, segs // L, initb, 0)

    def row(r, c):
        pltpu.sync_copy(mp_hbm.at[pl.ds(r * NSEG + sbase, segs)], tmp_v)

        def upd(j, cc):
            sl = pl.ds(j * L, L)
            acc_v[sl] = jnp.maximum(acc_v[sl], tmp_v[sl])
            return cc

        lax.fori_loop(0, segs // L, upd, 0)
        return c

    lax.fori_loop(0, NW, row, 0)
    pltpu.sync_copy(acc_v, m_hbm.at[pl.ds(sbase, segs)])


def _compare_body(
    yg0_hbm, yg1_hbm, yg2_hbm, yg3_hbm, idx_hbm, m_hbm, out_hbm,
    y0_v, y1_v, i0_v, i1_v, o0_v, o1_v, m_v,
    sy0, sy1, si0, si1, so0, so1,
):
    wid = _wid()
    base = wid * CHUNK
    ygs = (yg0_hbm, yg1_hbm, yg2_hbm, yg3_hbm)
    gsel = wid // (NW // G)
    lbase = (wid % (NW // G)) * CHUNK
    ybuf = (y0_v, y1_v)
    ibuf = (i0_v, i1_v)
    obuf = (o0_v, o1_v)
    ysem = (sy0, sy1)
    isem = (si0, si1)
    osem = (so0, so1)
    pltpu.sync_copy(m_hbm, m_v)
    one = jnp.full((L,), 1.0, jnp.float32)
    zero = jnp.full((L,), 0.0, jnp.float32)

    def start_in(b, p):
        off = base + b * BLK
        loff = lbase + b * BLK
        for gg in range(G):
            def _st(gg=gg):
                pltpu.async_copy(ygs[gg].at[pl.ds(loff, BLK)], ybuf[p], ysem[p])
                return None
            pl.when(gsel == gg)(_st)
        pltpu.async_copy(idx_hbm.at[pl.ds(off, BLK)], ibuf[p], isem[p])

    def wait_in(b, p):
        off = base + b * BLK
        loff = lbase + b * BLK
        for gg in range(G):
            def _wt(gg=gg):
                pltpu.make_async_copy(ygs[gg].at[pl.ds(loff, BLK)], ybuf[p], ysem[p]).wait()
                return None
            pl.when(gsel == gg)(_wt)
        pltpu.make_async_copy(idx_hbm.at[pl.ds(off, BLK)], ibuf[p], isem[p]).wait()

    def start_out(b, p):
        off = base + b * BLK
        pltpu.async_copy(obuf[p], out_hbm.at[pl.ds(off, BLK)], osem[p])

    def wait_out(b, p):
        off = base + b * BLK
        pltpu.make_async_copy(obuf[p], out_hbm.at[pl.ds(off, BLK)], osem[p]).wait()

    def compute(p):
        yb = ybuf[p]
        ib = ibuf[p]
        ob = obuf[p]
        lane = lax.iota(jnp.int32, L)

        def span(sp, cc):
            sbl = sp * SPAN
            pos = jnp.full((L,), sbl, jnp.int32) + lane * L
            starts = plsc.load_gather(ib, [pos])
            ends = plsc.load_gather(ib, [pos + (L - 1)])
            sa = starts[0]
            sz = ends[L - 1]

            def cmp_range(lo, hi, mvec):
                def body(i, c):
                    sl = pl.ds(sbl + i * L, L)
                    ob[sl] = jnp.where(yb[sl] == mvec, one, zero)
                    return c

                lax.fori_loop(lo, hi, body, 0)

            def span_fast():
                mvec = plsc.load_gather(m_v, [starts])
                for i in range(SPAN // L):
                    sl = pl.ds(sbl + i * L, L)
                    ob[sl] = jnp.where(yb[sl] == mvec, one, zero)

            def span_slow():
                kb = plsc.all_reduce_ffs(ends != jnp.full((L,), sa, jnp.int32))[0]
                kcr = plsc.all_reduce_ffs(
                    lax.rev(starts, (0,)) != jnp.full((L,), sz, jnp.int32)
                )[0]
                kc = (L - 1) - kcr
                cmp_range(0, kb, plsc.load_gather(m_v, [jnp.full((L,), sa, jnp.int32)]))

                def body(i, c):
                    sl = pl.ds(sbl + i * L, L)
                    mv = plsc.load_gather(m_v, [ib[sl]])
                    ob[sl] = jnp.where(yb[sl] == mv, one, zero)
                    return c

                lax.fori_loop(kb, kc + 1, body, 0)
                cmp_range(
                    kc + 1,
                    SPAN // L,
                    plsc.load_gather(m_v, [jnp.full((L,), sz, jnp.int32)]),
                )

            lax.cond(sa == sz, span_fast, span_slow)
            return cc

        lax.fori_loop(0, NSPAN, span, 0)

    start_in(0, 0)

    def outer(b2, c):
        for p in (0, 1):
            b = b2 * 2 + p
            pl.when(b + 1 < NBLK)(lambda: start_in(b + 1, 1 - p))
            wait_in(b, p)
            pl.when(b >= 2)(lambda: wait_out(b - 2, p))
            compute(p)
            start_out(b, p)
        return c

    lax.fori_loop(0, NBLK // 2, outer, 0)
    wait_out(NBLK - 2, 0)
    wait_out(NBLK - 1, 1)


_SEG_SCRATCH = [
    pltpu.VMEM((BLK,), jnp.float32),
    pltpu.VMEM((BLK,), jnp.float32),
    pltpu.VMEM((BLK,), jnp.int32),
    pltpu.VMEM((BLK,), jnp.int32),
    pltpu.VMEM((NSEG + L,), jnp.float32),
    pltpu.VMEM((L,), jnp.float32),
    pltpu.SemaphoreType.DMA,
    pltpu.SemaphoreType.DMA,
    pltpu.SemaphoreType.DMA,
    pltpu.SemaphoreType.DMA,
]

_seg_max_calls = [
    functools.partial(
        pl.kernel,
        out_type=jax.ShapeDtypeStruct((NW * NSEG,), jnp.float32),
        mesh=_mesh(),
        compiler_params=pltpu.CompilerParams(needs_layout_passes=False),
        scratch_types=_SEG_SCRATCH,
    )(_make_seg_max_body(g * GE, g == 0))
    for g in range(G)
]

_merge = functools.partial(
    pl.kernel,
    out_type=jax.ShapeDtypeStruct((NSEG,), jnp.float32),
    mesh=_mesh(),
    compiler_params=pltpu.CompilerParams(needs_layout_passes=False),
    scratch_types=[
        pltpu.VMEM((NSEG // NW,), jnp.float32),
        pltpu.VMEM((NSEG // NW,), jnp.float32),
    ],
)(_merge_body)

_compare = functools.partial(
    pl.kernel,
    out_type=jax.ShapeDtypeStruct((N,), jnp.float32),
    mesh=_mesh(),
    compiler_params=pltpu.CompilerParams(needs_layout_passes=False),
    scratch_types=[
        pltpu.VMEM((BLK,), jnp.float32),
        pltpu.VMEM((BLK,), jnp.float32),
        pltpu.VMEM((BLK,), jnp.int32),
        pltpu.VMEM((BLK,), jnp.int32),
        pltpu.VMEM((BLK,), jnp.float32),
        pltpu.VMEM((BLK,), jnp.float32),
        pltpu.VMEM((NSEG,), jnp.float32),
        pltpu.SemaphoreType.DMA,
        pltpu.SemaphoreType.DMA,
        pltpu.SemaphoreType.DMA,
        pltpu.SemaphoreType.DMA,
        pltpu.SemaphoreType.DMA,
        pltpu.SemaphoreType.DMA,
    ],
)(_compare_body)


def _gumbel_range(kd, start, size):
    c2 = lax.iota(jnp.uint32, size) + jnp.uint32(start)
    c1 = jnp.zeros((size,), jnp.uint32)
    b1, b2 = threefry2x32_p.bind(kd[0], kd[1], c1, c2)
    bits = b1 ^ b2
    fb = lax.shift_right_logical(bits, jnp.uint32(9))
    fb = lax.bitwise_or(fb, jnp.uint32(np.array(1.0, np.float32).view(np.uint32)))
    floats = lax.bitcast_convert_type(fb, jnp.float32) - jnp.float32(1.0)
    tiny = jnp.float32(np.finfo(np.float32).tiny)
    u = lax.max(tiny, floats * (jnp.float32(1.0) - tiny) + tiny)
    return -jnp.log(-jnp.log(u))


def kernel(logits, index):
    gkey = jax.random.fold_in(jax.random.key(0), 1)
    kd = jax.random.key_data(gkey)
    ys = [
        logits[g * GE:(g + 1) * GE] + _gumbel_range(kd, g * GE, GE)
        for g in range(G)
    ]
    mp = _seg_max_calls[0](ys[0], index)
    for g in range(1, G):
        mp = _seg_max_calls[g](ys[g], index, mp)
    m = _merge(mp)
    return _compare(ys[0], ys[1], ys[2], ys[3], index, m)


# speedup vs baseline: 1.0433x; 1.0433x over previous
import functools

import jax
import jax.numpy as jnp
from jax import lax
from jax.experimental import pallas as pl
from jax.experimental.pallas import tpu as pltpu
from jax.experimental.pallas import tpu_sc as plsc

NSEG = 16384
N = NSEG * 1000

NC = 2
NS = 16
NW = NC * NS
L = 16

CHUNK = N // NW
BLK = 12800
SPAN = 256
NBLK = CHUNK // BLK
NSPAN = BLK // SPAN
NEG_INF = float("-inf")


def _wid():
    return lax.axis_index("c") * NS + lax.axis_index("s")


def _mesh():
    return plsc.VectorSubcoreMesh(
        core_axis_name="c", subcore_axis_name="s", num_cores=NC, num_subcores=NS
    )


def _seg_max_body(
    y_hbm, idx_hbm, mp_hbm, y0_v, y1_v, i0_v, i1_v, tab_v, yscr_v, sy0, sy1, si0, si1
):
    wid = _wid()
    base = wid * CHUNK
    lane = lax.iota(jnp.int32, L)
    ybuf = (y0_v, y1_v)
    ibuf = (i0_v, i1_v)
    ysem = (sy0, sy1)
    isem = (si0, si1)

    def init(i, c):
        tab_v[pl.ds(i * L, L)] = jnp.full((L,), NEG_INF, jnp.float32)
        return c

    lax.fori_loop(0, NSEG // L + 1, init, 0)

    def start_in(b, p):
        off = base + b * BLK
        pltpu.async_copy(y_hbm.at[pl.ds(off, BLK)], ybuf[p], ysem[p])
        pltpu.async_copy(idx_hbm.at[pl.ds(off, BLK)], ibuf[p], isem[p])

    def wait_in(b, p):
        off = base + b * BLK
        pltpu.make_async_copy(y_hbm.at[pl.ds(off, BLK)], ybuf[p], ysem[p]).wait()
        pltpu.make_async_copy(idx_hbm.at[pl.ds(off, BLK)], ibuf[p], isem[p]).wait()

    def rmw(ixvec, val_vec, mask):
        t = plsc.load_gather(tab_v, [ixvec])
        plsc.store_scatter(tab_v, [ixvec], jnp.maximum(t, val_vec), mask=mask)

    def flush(cs, acc):
        m = jnp.max(acc)
        rmw(jnp.full((L,), cs, jnp.int32), jnp.full((L,), m, jnp.float32), lane == 0)

    def compute(p, cs, acc):
        yb = ybuf[p]
        ib = ibuf[p]

        def kogge(vb):
            ix = ib[pl.ds(vb, L)]
            yy = yb[pl.ds(vb, L)]
            vbv = jnp.full((L,), vb, jnp.int32)
            for d in (1, 2, 4, 8):
                ok0 = lane >= d
                yscr_v[...] = yy
                ys = plsc.load_gather(yscr_v, [lane - d], mask=ok0)
                ixs = plsc.load_gather(ib, [vbv + (lane - d)], mask=ok0)
                ok = ok0 & (ix == ixs)
                yy = jnp.where(ok, jnp.maximum(yy, ys), yy)
            nmask = lane < (L - 1)
            ixn = plsc.load_gather(ib, [vbv + (lane + 1)], mask=nmask)
            end = (lane == L - 1) | (nmask & (ix != ixn))
            rmw(ix, yy, end)

        def span(sp, carry):
            cs, acc = carry
            sbl = sp * SPAN
            pos = jnp.full((L,), sbl, jnp.int32) + lane * L
            starts = plsc.load_gather(ib, [pos])
            ends = plsc.load_gather(ib, [pos + (L - 1)])
            sa = starts[0]
            sz = ends[L - 1]

            def uniform_case(cs, acc):
                v = yb[pl.ds(sbl, L)]
                for i in range(1, SPAN // L):
                    v = jnp.maximum(v, yb[pl.ds(sbl + i * L, L)])

                def same():
                    return cs, jnp.maximum(acc, v)

                def diff():
                    flush(cs, acc)
                    return sa, v

                return lax.cond(sa == cs, same, diff)

            def slow_case(cs, acc):
                flush(cs, acc)
                kb = plsc.all_reduce_ffs(ends != jnp.full((L,), sa, jnp.int32))[0]
                kcr = plsc.all_reduce_ffs(
                    lax.rev(starts, (0,)) != jnp.full((L,), sz, jnp.int32)
                )[0]
                kc = (L - 1) - kcr

                def lstep(k, a):
                    return jnp.maximum(a, yb[pl.ds(sbl + k * L, L)])

                lacc = lax.fori_loop(0, kb, lstep, jnp.full((L,), NEG_INF, jnp.float32))
                flush(sa, lacc)

                def kstep(k, c):
                    kogge(sbl + k * L)
                    return c

                lax.fori_loop(kb, kc + 1, kstep, 0)
                racc = lax.fori_loop(
                    kc + 1, SPAN // L, lstep, jnp.full((L,), NEG_INF, jnp.float32)
                )
                return sz, racc

            return lax.cond(sa == sz, uniform_case, slow_case, cs, acc)

        return lax.fori_loop(0, NSPAN, span, (cs, acc))

    start_in(0, 0)

    def outer(b2, carry):
        cs, acc = carry
        for p in (0, 1):
            b = b2 * 2 + p
            pl.when(b + 1 < NBLK)(lambda: start_in(b + 1, 1 - p))
            wait_in(b, p)
            cs, acc = compute(p, cs, acc)
        return cs, acc

    cs0 = jnp.int32(NSEG)
    acc0 = jnp.full((L,), NEG_INF, jnp.float32)
    cs, acc = lax.fori_loop(0, NBLK // 2, outer, (cs0, acc0))
    flush(cs, acc)
    pltpu.sync_copy(tab_v.at[pl.ds(0, NSEG)], mp_hbm.at[pl.ds(wid * NSEG, NSEG)])


def _merge_body(mp_hbm, m_hbm, tmp_v, acc_v):
    wid = _wid()
    segs = NSEG // NW
    sbase = wid * segs

    def initb(j, c):
        acc_v[pl.ds(j * L, L)] = jnp.full((L,), NEG_INF, jnp.float32)
        return c

    lax.fori_loop(0, segs // L, initb, 0)

    def row(r, c):
        pltpu.sync_copy(mp_hbm.at[pl.ds(r * NSEG + sbase, segs)], tmp_v)

        def upd(j, cc):
            sl = pl.ds(j * L, L)
            acc_v[sl] = jnp.maximum(acc_v[sl], tmp_v[sl])
            return cc

        lax.fori_loop(0, segs // L, upd, 0)
        return c

    lax.fori_loop(0, NW, row, 0)
    pltpu.sync_copy(acc_v, m_hbm.at[pl.ds(sbase, segs)])


def _compare_body(
    y_hbm, idx_hbm, m_hbm, out_hbm,
    y0_v, y1_v, i0_v, i1_v, o0_v, o1_v, m_v,
    sy0, sy1, si0, si1, so0, so1,
):
    wid = _wid()
    base = wid * CHUNK
    ybuf = (y0_v, y1_v)
    ibuf = (i0_v, i1_v)
    obuf = (o0_v, o1_v)
    ysem = (sy0, sy1)
    isem = (si0, si1)
    osem = (so0, so1)
    pltpu.sync_copy(m_hbm, m_v)
    one = jnp.full((L,), 1.0, jnp.float32)
    zero = jnp.full((L,), 0.0, jnp.float32)

    def start_in(b, p):
        off = base + b * BLK
        pltpu.async_copy(y_hbm.at[pl.ds(off, BLK)], ybuf[p], ysem[p])
        pltpu.async_copy(idx_hbm.at[pl.ds(off, BLK)], ibuf[p], isem[p])

    def wait_in(b, p):
        off = base + b * BLK
        pltpu.make_async_copy(y_hbm.at[pl.ds(off, BLK)], ybuf[p], ysem[p]).wait()
        pltpu.make_async_copy(idx_hbm.at[pl.ds(off, BLK)], ibuf[p], isem[p]).wait()

    def start_out(b, p):
        off = base + b * BLK
        pltpu.async_copy(obuf[p], out_hbm.at[pl.ds(off, BLK)], osem[p])

    def wait_out(b, p):
        off = base + b * BLK
        pltpu.make_async_copy(obuf[p], out_hbm.at[pl.ds(off, BLK)], osem[p]).wait()

    def compute(p):
        yb = ybuf[p]
        ib = ibuf[p]
        ob = obuf[p]
        lane = lax.iota(jnp.int32, L)

        def span(sp, cc):
            sbl = sp * SPAN
            pos = jnp.full((L,), sbl, jnp.int32) + lane * L
            starts = plsc.load_gather(ib, [pos])
            ends = plsc.load_gather(ib, [pos + (L - 1)])
            sa = starts[0]
            sz = ends[L - 1]

            def cmp_range(lo, hi, mvec):
                def body(i, c):
                    sl = pl.ds(sbl + i * L, L)
                    ob[sl] = jnp.where(yb[sl] == mvec, one, zero)
                    return c

                lax.fori_loop(lo, hi, body, 0)

            def span_fast():
                mvec = plsc.load_gather(m_v, [starts])
                for i in range(SPAN // L):
                    sl = pl.ds(sbl + i * L, L)
                    ob[sl] = jnp.where(yb[sl] == mvec, one, zero)

            def span_slow():
                kb = plsc.all_reduce_ffs(ends != jnp.full((L,), sa, jnp.int32))[0]
                kcr = plsc.all_reduce_ffs(
                    lax.rev(starts, (0,)) != jnp.full((L,), sz, jnp.int32)
                )[0]
                kc = (L - 1) - kcr
                cmp_range(0, kb, plsc.load_gather(m_v, [jnp.full((L,), sa, jnp.int32)]))

                def body(i, c):
                    sl = pl.ds(sbl + i * L, L)
                    mv = plsc.load_gather(m_v, [ib[sl]])
                    ob[sl] = jnp.where(yb[sl] == mv, one, zero)
                    return c

                lax.fori_loop(kb, kc + 1, body, 0)
                cmp_range(
                    kc + 1,
                    SPAN // L,
                    plsc.load_gather(m_v, [jnp.full((L,), sz, jnp.int32)]),
                )

            lax.cond(sa == sz, span_fast, span_slow)
            return cc

        lax.fori_loop(0, NSPAN, span, 0)

    start_in(0, 0)

    def outer(b2, c):
        for p in (0, 1):
            b = b2 * 2 + p
            pl.when(b + 1 < NBLK)(lambda: start_in(b + 1, 1 - p))
            wait_in(b, p)
            pl.when(b >= 2)(lambda: wait_out(b - 2, p))
            compute(p)
            start_out(b, p)
        return c

    lax.fori_loop(0, NBLK // 2, outer, 0)
    wait_out(NBLK - 2, 0)
    wait_out(NBLK - 1, 1)


_seg_max = functools.partial(
    pl.kernel,
    out_type=jax.ShapeDtypeStruct((NW * NSEG,), jnp.float32),
    mesh=_mesh(),
    compiler_params=pltpu.CompilerParams(needs_layout_passes=False),
    scratch_types=[
        pltpu.VMEM((BLK,), jnp.float32),
        pltpu.VMEM((BLK,), jnp.float32),
        pltpu.VMEM((BLK,), jnp.int32),
        pltpu.VMEM((BLK,), jnp.int32),
        pltpu.VMEM((NSEG + L,), jnp.float32),
        pltpu.VMEM((L,), jnp.float32),
        pltpu.SemaphoreType.DMA,
        pltpu.SemaphoreType.DMA,
        pltpu.SemaphoreType.DMA,
        pltpu.SemaphoreType.DMA,
    ],
)(_seg_max_body)

_merge = functools.partial(
    pl.kernel,
    out_type=jax.ShapeDtypeStruct((NSEG,), jnp.float32),
    mesh=_mesh(),
    compiler_params=pltpu.CompilerParams(needs_layout_passes=False),
    scratch_types=[
        pltpu.VMEM((NSEG // NW,), jnp.float32),
        pltpu.VMEM((NSEG // NW,), jnp.float32),
    ],
)(_merge_body)

_compare = functools.partial(
    pl.kernel,
    out_type=jax.ShapeDtypeStruct((N,), jnp.float32),
    mesh=_mesh(),
    compiler_params=pltpu.CompilerParams(needs_layout_passes=False),
    scratch_types=[
        pltpu.VMEM((BLK,), jnp.float32),
        pltpu.VMEM((BLK,), jnp.float32),
        pltpu.VMEM((BLK,), jnp.int32),
        pltpu.VMEM((BLK,), jnp.int32),
        pltpu.VMEM((BLK,), jnp.float32),
        pltpu.VMEM((BLK,), jnp.float32),
        pltpu.VMEM((NSEG,), jnp.float32),
        pltpu.SemaphoreType.DMA,
        pltpu.SemaphoreType.DMA,
        pltpu.SemaphoreType.DMA,
        pltpu.SemaphoreType.DMA,
        pltpu.SemaphoreType.DMA,
        pltpu.SemaphoreType.DMA,
    ],
)(_compare_body)


def kernel(logits, index):
    gkey = jax.random.fold_in(jax.random.key(0), 1)
    z = jax.random.gumbel(gkey, logits.shape, logits.dtype)
    y = logits + z
    mp = _seg_max(y, index)
    m = _merge(mp)
    return _compare(y, index, m)
